# baseline (speedup 1.0000x reference)
import jax
import jax.numpy as jnp
from jax import lax
from jax.experimental import pallas as pl
from jax.experimental.pallas import tpu as pltpu

_BC = 48


def _pool_compute(x):
    rm = jnp.maximum(x[:, :-1, :], x[:, 1:, :])
    return jnp.maximum(rm[:, :, :-1], rm[:, :, 1:])


def _make_body(N, C, H, W):
    n_cb = C // _BC
    steps = N * n_cb

    def body(x_hbm, o_hbm, xbuf, obuf, in_sem, out_sem):
        s = pl.program_id(0)
        slot = lax.rem(s, 2)

        def in_copy(step, buf_slot):
            n = lax.div(step, n_cb)
            cb = lax.rem(step, n_cb)
            return pltpu.make_async_copy(
                x_hbm.at[n, pl.ds(cb * _BC, _BC)],
                xbuf.at[buf_slot],
                in_sem.at[buf_slot],
            )

        def out_copy(step, buf_slot):
            n = lax.div(step, n_cb)
            cb = lax.rem(step, n_cb)
            return pltpu.make_async_copy(
                obuf.at[buf_slot],
                o_hbm.at[n, pl.ds(cb * _BC, _BC)],
                out_sem.at[buf_slot],
            )

        @pl.when(s == 0)
        def _():
            in_copy(0, 0).start()

        in_copy(s, slot).wait()

        @pl.when(s >= 2)
        def _():
            out_copy(s - 2, slot).wait()

        obuf[slot] = _pool_compute(xbuf[slot])

        @pl.when(s + 1 < steps)
        def _():
            in_copy(s + 1, 1 - slot).start()

        out_copy(s, slot).start()

        @pl.when(s == steps - 1)
        def _():
            if steps > 1:
                out_copy(s - 1, 1 - slot).wait()
            out_copy(s, slot).wait()

    return body, steps


def kernel(x):
    N, C, H, W = x.shape
    body, steps = _make_body(N, C, H, W)
    return pl.pallas_call(
        body,
        grid=(steps,),
        in_specs=[pl.BlockSpec(memory_space=pl.ANY)],
        out_specs=pl.BlockSpec(memory_space=pl.ANY),
        out_shape=jax.ShapeDtypeStruct((N, C, H - 1, W - 1), x.dtype),
        scratch_shapes=[
            pltpu.VMEM((2, _BC, H, W), jnp.float32),
            pltpu.VMEM((2, _BC, H - 1, W - 1), jnp.float32),
            pltpu.SemaphoreType.DMA((2,)),
            pltpu.SemaphoreType.DMA((2,)),
        ],
        compiler_params=pltpu.CompilerParams(
            dimension_semantics=("arbitrary",),
            vmem_limit_bytes=100 * 1024 * 1024,
        ),
    )(x)


# speedup vs baseline: 1.2258x; 1.2258x over previous
import jax
import jax.numpy as jnp
from jax import lax
from jax.experimental import pallas as pl
from jax.experimental.pallas import tpu as pltpu

_BC = 32
_NS = 3


def _pool_compute(x):
    rm = jnp.maximum(x[:, :-1, :], x[:, 1:, :])
    return jnp.maximum(rm[:, :, :-1], rm[:, :, 1:])


def _make_body(N, C, H, W):
    n_cb = C // _BC
    steps = N * n_cb

    def body(x_hbm, o_hbm, xbuf, obuf, in_sem, out_sem):
        s = pl.program_id(0)
        slot = lax.rem(s, _NS)

        def in_copy(step):
            n = lax.div(step, n_cb)
            cb = lax.rem(step, n_cb)
            return pltpu.make_async_copy(
                x_hbm.at[n, pl.ds(cb * _BC, _BC)],
                xbuf.at[lax.rem(step, _NS)],
                in_sem.at[lax.rem(step, _NS)],
            )

        def out_copy(step):
            n = lax.div(step, n_cb)
            cb = lax.rem(step, n_cb)
            return pltpu.make_async_copy(
                obuf.at[lax.rem(step, _NS)],
                o_hbm.at[n, pl.ds(cb * _BC, _BC)],
                out_sem.at[lax.rem(step, _NS)],
            )

        @pl.when(s == 0)
        def _():
            for k in range(min(_NS, steps)):
                in_copy(k).start()

        in_copy(s).wait()

        @pl.when(s >= _NS)
        def _():
            out_copy(s - _NS).wait()

        obuf[slot] = _pool_compute(xbuf[slot])

        out_copy(s).start()

        @pl.when(s + _NS < steps)
        def _():
            in_copy(s + _NS).start()

        @pl.when(s == steps - 1)
        def _():
            for k in range(min(_NS - 1, steps - 1), -1, -1):
                out_copy(s - k).wait()

    return body, steps


def kernel(x):
    N, C, H, W = x.shape
    body, steps = _make_body(N, C, H, W)
    return pl.pallas_call(
        body,
        grid=(steps,),
        in_specs=[pl.BlockSpec(memory_space=pl.ANY)],
        out_specs=pl.BlockSpec(memory_space=pl.ANY),
        out_shape=jax.ShapeDtypeStruct((N, C, H - 1, W - 1), x.dtype),
        scratch_shapes=[
            pltpu.VMEM((_NS, _BC, H, W), jnp.float32),
            pltpu.VMEM((_NS, _BC, H - 1, W - 1), jnp.float32),
            pltpu.SemaphoreType.DMA((_NS,)),
            pltpu.SemaphoreType.DMA((_NS,)),
        ],
        compiler_params=pltpu.CompilerParams(
            dimension_semantics=("arbitrary",),
            vmem_limit_bytes=100 * 1024 * 1024,
        ),
    )(x)


# speedup vs baseline: 1.4376x; 1.1727x over previous
import jax
import jax.numpy as jnp
from jax.experimental import pallas as pl
from jax.experimental.pallas import tpu as pltpu

_BC = 48


def _pool_body(x_ref, o_ref):
    x = x_ref[...]
    rm = jnp.maximum(x[:, :-1, :], x[:, 1:, :])
    o_ref[...] = jnp.maximum(rm[:, :, :-1], rm[:, :, 1:])


def kernel(x):
    N, C, H, W = x.shape
    nc = N * C
    xf = x.reshape(nc, H, W)
    out = pl.pallas_call(
        _pool_body,
        grid=(nc // _BC,),
        in_specs=[pl.BlockSpec((_BC, H, W), lambda i: (i, 0, 0))],
        out_specs=pl.BlockSpec((_BC, H - 1, W - 1), lambda i: (i, 0, 0)),
        out_shape=jax.ShapeDtypeStruct((nc, H - 1, W - 1), x.dtype),
        compiler_params=pltpu.CompilerParams(
            dimension_semantics=("parallel",),
            vmem_limit_bytes=100 * 1024 * 1024,
        ),
    )(xf)
    return out.reshape(N, C, H - 1, W - 1)
